# baseline (speedup 1.0000x reference)
import functools

import jax
import jax.numpy as jnp
from jax import lax
from jax.experimental import pallas as pl
from jax.experimental.pallas import tpu as pltpu
from jax.experimental.pallas import tpu_sc as plsc

N = 10000
NP = 10240
E = 160000
EPT = 10240
EPAD = 16 * EPT
CHUNK = 128
NCHUNK = EPT // CHUNK
H = 512
OUT = 256
NE = 8
G = 64
BM = 640
NT = NP // BM
CW = 64
NCB = 8

f32 = jnp.float32
bf16 = jnp.bfloat16
i32 = jnp.int32


def _bdot(a, b):
    return jnp.dot(a.astype(bf16), b.astype(bf16), preferred_element_type=f32)


def _make_agg(nexp: int, ncb: int, with_counts: bool, cw: int = CW):
    mesh = plsc.VectorSubcoreMesh(core_axis_name="c", subcore_axis_name="s")
    out_type = [jax.ShapeDtypeStruct((nexp * ncb, NP, cw), f32)]
    if with_counts:
        out_type.append(jax.ShapeDtypeStruct((NP, cw), f32))

    def body(hv, src4, dstT, *refs):
        if with_counts:
            (sums, deg, srcT, dstv, rows, zbuf, acc,
             gs0, gs1, gs2, gs3, ss0, ss1, ss2, ss3) = refs
        else:
            (sums, srcT, dstv, rows, zbuf, acc,
             gs0, gs1, gs2, gs3, ss0, ss1, ss2, ss3) = refs
        gsems = (gs0, gs1, gs2, gs3)
        ssems = (ss0, ss1, ss2, ss3)
        c = lax.axis_index("c")
        s = lax.axis_index("s")
        row0 = s * 640

        z16 = jnp.zeros((16,), f32)

        def zz(i, _):
            for k in range(cw // 16):
                zbuf[i, pl.ds(k * 16, 16)] = z16
            return 0
        lax.fori_loop(0, 64, zz, 0)

        pltpu.sync_copy(dstT.at[s], dstv)

        def zero_acc():
            for m in range(10):
                pltpu.async_copy(zbuf, acc.at[pl.ds(row0 + 64 * m, 64)], gs0)
            for m in range(10):
                pltpu.make_async_copy(zbuf, acc.at[pl.ds(row0 + 64 * m, 64)],
                                      gs0).wait()

        def start_gather(e, j, slot):
            pltpu.async_copy(hv.at[e].at[srcT.at[j]], rows.at[slot], gsems[slot])

        def wait_gather(e, j, slot):
            pltpu.make_async_copy(hv.at[e].at[srcT.at[j]], rows.at[slot],
                                  gsems[slot]).wait()

        def start_scatter(j, slot):
            pltpu.async_copy(rows.at[slot], acc.at[dstv.at[j]], ssems[slot],
                             add=True)

        def wait_scatter(j, slot):
            pltpu.make_async_copy(rows.at[slot], acc.at[dstv.at[j]],
                                  ssems[slot]).wait()

        def do_block(e, cb):
            zero_acc()
            plsc.subcore_barrier()
            start_gather(e, 0, 0)
            start_gather(e, 1, 1)

            def chunk_body(j, _):
                for k in range(4):
                    c = 4 * j + k
                    wait_gather(e, c, k)
                    start_scatter(c, k)
                    sd = (k + 2) % 4
                    if k < 2:
                        @pl.when(j > 0)
                        def _():
                            wait_scatter(c - 2, sd)
                            start_gather(e, c + 2, sd)

                        @pl.when(j == 0)
                        def _():
                            start_gather(e, c + 2, sd)
                    else:
                        @pl.when(j < NCHUNK // 4 - 1)
                        def _():
                            wait_scatter(c - 2, sd)
                            start_gather(e, c + 2, sd)
                return 0
            lax.fori_loop(0, NCHUNK // 4, chunk_body, 0)
            wait_scatter(NCHUNK - 4, 0)
            wait_scatter(NCHUNK - 3, 1)
            wait_scatter(NCHUNK - 2, 2)
            wait_scatter(NCHUNK - 1, 3)
            plsc.subcore_barrier()
            pltpu.sync_copy(acc.at[pl.ds(row0, 640)],
                            sums.at[e * ncb + cb, pl.ds(row0, 640)])
            plsc.subcore_barrier()

        for k in range(ncb // 2):
            cb = c * (ncb // 2) + k
            pltpu.sync_copy(src4.at[cb, s], srcT)
            lax.fori_loop(0, nexp, lambda e, _: (do_block(e, cb), 0)[1], 0)

        if with_counts:
            @pl.when(c == 0)
            def _():
                one16 = jnp.ones((16,), f32)

                def oo(i, _):
                    for k in range(cw // 16):
                        rows[0, i, pl.ds(k * 16, 16)] = one16
                    return 0
                lax.fori_loop(0, 128, oo, 0)
                zero_acc()
                plsc.subcore_barrier()

                def cnt_body(i, _):
                    pltpu.sync_copy(rows.at[0], acc.at[dstv.at[i]], add=True)
                    return 0
                lax.fori_loop(0, NCHUNK, cnt_body, 0)
                plsc.subcore_barrier()
                pltpu.sync_copy(acc.at[pl.ds(row0, 640)],
                                deg.at[pl.ds(row0, 640)])

    kern = pl.kernel(
        body,
        out_type=out_type,
        mesh=mesh,
        compiler_params=pltpu.CompilerParams(use_tc_tiling_on_sc=False),
        scratch_types=[
            pltpu.VMEM((NCHUNK, 128), i32),
            pltpu.VMEM((NCHUNK, 128), i32),
            pltpu.VMEM((4, CHUNK, cw), f32),
            pltpu.VMEM((64, cw), f32),
            pltpu.VMEM_SHARED((NP, cw), f32),
            pltpu.SemaphoreType.DMA,
            pltpu.SemaphoreType.DMA,
            pltpu.SemaphoreType.DMA,
            pltpu.SemaphoreType.DMA,
            pltpu.SemaphoreType.DMA,
            pltpu.SemaphoreType.DMA,
            pltpu.SemaphoreType.DMA,
            pltpu.SemaphoreType.DMA,
        ],
    )
    return kern


def _counts_body(batch_ref, sf_ref):
    b = batch_ref[...]
    oh = (b == lax.broadcasted_iota(i32, (NP, G), 1)).astype(f32)
    counts = jnp.sum(oh, axis=0, keepdims=True)
    lg = jnp.log1p(counts)
    sf_ref[...] = jnp.sum(oh * lg, axis=1, keepdims=True)


def _enc_router_body(x_ref, sf_ref, Wenc_ref, benc_ref, Wr1_ref, wr1s_ref,
                     br1_ref, Wr2_ref, br2_ref, h_ref, probs_ref):
    h = jnp.maximum(jnp.dot(x_ref[...], Wenc_ref[...],
                            preferred_element_type=f32) + benc_ref[0], 0.0)
    r = jnp.dot(h, Wr1_ref[...], preferred_element_type=f32)
    r = jnp.maximum(r + sf_ref[...] * wr1s_ref[0] + br1_ref[0], 0.0)
    lg = jnp.dot(r, Wr2_ref[...], preferred_element_type=f32) + br2_ref[0]
    m = jnp.max(lg, axis=1, keepdims=True)
    p = jnp.exp(lg - m)
    probs_ref[...] = p / jnp.sum(p, axis=1, keepdims=True)
    h_ref[...] = h


def _layer_body(h_ref, A_ref, deg_ref, Ws_ref, Wn_ref, b_ref, *refs, relu,
                fuse_next):
    dinv = 1.0 / jnp.maximum(deg_ref[:, 0:1], 1.0)
    z = _bdot(h_ref[0], Ws_ref[0])
    a = jnp.concatenate([A_ref[0, cb] for cb in range(NCB)], axis=1) * dinv
    z = z + _bdot(a, Wn_ref[0])
    z = z + b_ref[0]
    hh = jnp.maximum(z, 0.0) if relu else z
    if fuse_next:
        Wno_ref, o_ref, p_ref = refs
        o_ref[0] = hh
        p_ref[0] = _bdot(hh, Wno_ref[0])
    else:
        (o_ref,) = refs
        o_ref[0] = hh


def _out_body(h_ref, A_ref, deg_ref, probs_ref, Ws_ref, b_ref, o_ref):
    e = pl.program_id(1)
    dinv = 1.0 / jnp.maximum(deg_ref[:, 0:1], 1.0)
    z = _bdot(h_ref[0], Ws_ref[0])
    z = z + jnp.concatenate([A_ref[0, cb] for cb in range(A_ref.shape[1])],
                            axis=1) * dinv
    z = z + b_ref[0]
    oh = (lax.broadcasted_iota(i32, (1, NE), 1) == e).astype(f32)
    pe = jnp.sum(probs_ref[...] * oh, axis=1, keepdims=True)

    @pl.when(e == 0)
    def _():
        o_ref[...] = jnp.zeros_like(o_ref)
    o_ref[...] += pe * z


def _counts_call(batch2):
    return pl.pallas_call(
        _counts_body,
        out_shape=jax.ShapeDtypeStruct((NP, 1), f32),
    )(batch2)


def _enc_router_call(x2, sf, Wenc, benc, Wr1h, wr1s, br1, Wr2, br2):
    return pl.pallas_call(
        _enc_router_body,
        grid=(NT,),
        in_specs=[
            pl.BlockSpec((BM, 8), lambda t: (t, 0)),
            pl.BlockSpec((BM, 1), lambda t: (t, 0)),
            pl.BlockSpec((8, H), lambda t: (0, 0)),
            pl.BlockSpec((1, 1, H), lambda t: (0, 0, 0)),
            pl.BlockSpec((H, H), lambda t: (0, 0)),
            pl.BlockSpec((1, 1, H), lambda t: (0, 0, 0)),
            pl.BlockSpec((1, 1, H), lambda t: (0, 0, 0)),
            pl.BlockSpec((H, NE), lambda t: (0, 0)),
            pl.BlockSpec((1, 1, NE), lambda t: (0, 0, 0)),
        ],
        out_specs=[
            pl.BlockSpec((BM, H), lambda t: (t, 0)),
            pl.BlockSpec((BM, NE), lambda t: (t, 0)),
        ],
        out_shape=[
            jax.ShapeDtypeStruct((NP, H), f32),
            jax.ShapeDtypeStruct((NP, NE), f32),
        ],
    )(x2, sf, Wenc, benc, Wr1h, wr1s, br1, Wr2, br2)


def _layer_call(h, A, deg, Ws, Wn, b, *, shared_h, Wno=None):
    h3 = h[None] if shared_h else h
    hmap = (lambda e, t: (0, t, 0)) if shared_h else (lambda e, t: (e, t, 0))
    A4 = A.reshape(-1, NCB, NP, CW)
    amap = (lambda e, t: (0, 0, t, 0)) if A4.shape[0] == 1 else (lambda e, t: (e, 0, t, 0))
    fuse = Wno is not None
    in_specs = [
        pl.BlockSpec((1, BM, H), hmap),
        pl.BlockSpec((1, NCB, BM, CW), amap),
        pl.BlockSpec((BM, CW), lambda e, t: (t, 0)),
        pl.BlockSpec((1, H, H), lambda e, t: (e, 0, 0)),
        pl.BlockSpec((1, H, H), lambda e, t: (e, 0, 0)),
        pl.BlockSpec((1, 1, H), lambda e, t: (e, 0, 0)),
    ]
    args = [h3, A4, deg, Ws, Wn, b]
    out_specs = [pl.BlockSpec((1, BM, H), lambda e, t: (e, t, 0))]
    out_shape = [jax.ShapeDtypeStruct((NE, NP, H), f32)]
    if fuse:
        in_specs.append(pl.BlockSpec((1, H, OUT), lambda e, t: (e, 0, 0)))
        args.append(Wno)
        out_specs.append(pl.BlockSpec((1, BM, OUT), lambda e, t: (e, t, 0)))
        out_shape.append(jax.ShapeDtypeStruct((NE, NP, OUT), f32))
    res = pl.pallas_call(
        functools.partial(_layer_body, relu=True, fuse_next=fuse),
        grid=(NE, NT),
        in_specs=in_specs,
        out_specs=out_specs,
        out_shape=out_shape,
    )(*args)
    return res if fuse else res[0]


def _out_call(h2, A2p, deg, probs, Wso, bo):
    A4 = A2p.reshape(NE, OUT // 32, NP, 32)
    return pl.pallas_call(
        _out_body,
        grid=(NT, NE),
        in_specs=[
            pl.BlockSpec((1, BM, H), lambda t, e: (e, t, 0)),
            pl.BlockSpec((1, OUT // 32, BM, 32), lambda t, e: (e, 0, t, 0)),
            pl.BlockSpec((BM, CW), lambda t, e: (t, 0)),
            pl.BlockSpec((BM, NE), lambda t, e: (t, 0)),
            pl.BlockSpec((1, H, OUT), lambda t, e: (e, 0, 0)),
            pl.BlockSpec((1, 1, OUT), lambda t, e: (e, 0, 0)),
        ],
        out_specs=pl.BlockSpec((BM, OUT), lambda t, e: (t, 0)),
        out_shape=jax.ShapeDtypeStruct((NP, OUT), f32),
    )(h2, A4, deg, probs, Wso, bo)


@jax.jit
def _forward(x, edge_index, batch, W_enc, b_enc, Wr1, br1, Wr2, br2,
             Ws_h, Wn_h, b_h, Ws_o, Wn_o, b_o):
    src = edge_index[0]
    dst = edge_index[1]

    x2 = jnp.zeros((NP, 8), f32).at[:N, :6].set(x)
    batch2 = jnp.full((NP, 1), G, i32).at[:N, 0].set(batch)
    src_p = jnp.zeros((EPAD,), i32).at[:E].set(src)
    dst_p = jnp.full((EPAD,), N, i32).at[:E].set(dst)
    srcT = src_p.reshape(16, NCHUNK, 128)
    src4 = srcT[None] * NCB + jnp.arange(NCB, dtype=i32)[:, None, None, None]
    src4o = srcT[None] * (OUT // 32) + jnp.arange(OUT // 32, dtype=i32)[:, None, None, None]
    dstT = dst_p.reshape(16, NCHUNK, 128)

    Wenc = jnp.zeros((8, H), f32).at[:6].set(W_enc)
    benc = b_enc.reshape(1, 1, H)
    Wr1h = Wr1[:H]
    wr1s = Wr1[H].reshape(1, 1, H)
    br1r = br1.reshape(1, 1, H)
    br2r = br2.reshape(1, 1, NE)
    Ws0, Ws1 = Ws_h[:, 0], Ws_h[:, 1]
    Wn0, Wn1 = Wn_h[:, 0], Wn_h[:, 1]
    b0 = b_h[:, 0].reshape(NE, 1, H)
    b1 = b_h[:, 1].reshape(NE, 1, H)
    bo = b_o.reshape(NE, 1, OUT)

    agg1 = _make_agg(1, NCB, True)
    agg8 = _make_agg(NE, NCB, False)
    agg8o = _make_agg(NE, OUT // 32, False, cw=32)

    sf = _counts_call(batch2)
    h, probs = _enc_router_call(x2, sf, Wenc, benc, Wr1h, wr1s, br1r, Wr2, br2r)

    hv = h.reshape(1, NP * NCB, CW)
    A0, deg = agg1(hv, src4, dstT)
    h1 = _layer_call(h, A0, deg, Ws0, Wn0, b0, shared_h=True)

    h1v = h1.reshape(NE, NP * NCB, CW)
    (A1,) = agg8(h1v, src4, dstT)
    h2, p2 = _layer_call(h1, A1, deg, Ws1, Wn1, b1, shared_h=False, Wno=Wn_o)

    p2v = p2.reshape(NE, NP * (OUT // 32), 32)
    (A2p,) = agg8o(p2v, src4o, dstT)
    out = _out_call(h2, A2p, deg, probs, Ws_o, bo)
    return out[:N]


def kernel(x, edge_index, batch, W_enc, b_enc, Wr1, br1, Wr2, br2,
           Ws_h, Wn_h, b_h, Ws_o, Wn_o, b_o):
    return _forward(x, edge_index, batch, W_enc, b_enc, Wr1, br1, Wr2, br2,
                    Ws_h, Wn_h, b_h, Ws_o, Wn_o, b_o)


# speedup vs baseline: 1.1080x; 1.1080x over previous
import functools

import jax
import jax.numpy as jnp
from jax import lax
from jax.experimental import pallas as pl
from jax.experimental.pallas import tpu as pltpu
from jax.experimental.pallas import tpu_sc as plsc

N = 10000
NP = 10240
E = 160000
EPT = 10240
EPAD = 16 * EPT
CHUNK = 128
NCHUNK = EPT // CHUNK
H = 512
OUT = 256
NE = 8
G = 64
BM = 640
NT = NP // BM
CW = 64
NCB = 8

f32 = jnp.float32
bf16 = jnp.bfloat16
i32 = jnp.int32


def _bdot(a, b):
    return jnp.dot(a.astype(bf16), b.astype(bf16), preferred_element_type=f32)


def _make_agg(nexp: int, ncb: int, with_counts: bool, cw: int = CW,
              dt=f32):
    mesh = plsc.VectorSubcoreMesh(core_axis_name="c", subcore_axis_name="s")
    out_type = [jax.ShapeDtypeStruct((nexp * ncb, NP, cw), dt)]
    if with_counts:
        out_type.append(jax.ShapeDtypeStruct((NP, cw), f32))

    def body(hv, src4, dstT, *refs):
        if with_counts:
            (sums, deg, srcT, dstv, rows, zbuf, acc,
             gs0, gs1, gs2, gs3, ss0, ss1, ss2, ss3) = refs
        else:
            (sums, srcT, dstv, rows, zbuf, acc,
             gs0, gs1, gs2, gs3, ss0, ss1, ss2, ss3) = refs
        gsems = (gs0, gs1, gs2, gs3)
        ssems = (ss0, ss1, ss2, ss3)
        c = lax.axis_index("c")
        s = lax.axis_index("s")
        row0 = s * 640

        lanes = 16 if dt == f32 else 32
        zv = jnp.zeros((lanes,), dt)

        def zz(i, _):
            for k in range(cw // lanes):
                zbuf[i, pl.ds(k * lanes, lanes)] = zv
            return 0
        lax.fori_loop(0, 64, zz, 0)

        pltpu.sync_copy(dstT.at[s], dstv)

        def zero_acc():
            for m in range(10):
                pltpu.async_copy(zbuf, acc.at[pl.ds(row0 + 64 * m, 64)], gs0)
            for m in range(10):
                pltpu.make_async_copy(zbuf, acc.at[pl.ds(row0 + 64 * m, 64)],
                                      gs0).wait()

        def start_gather(e, j, slot):
            pltpu.async_copy(hv.at[e].at[srcT.at[j]], rows.at[slot], gsems[slot])

        def wait_gather(e, j, slot):
            pltpu.make_async_copy(hv.at[e].at[srcT.at[j]], rows.at[slot],
                                  gsems[slot]).wait()

        def start_scatter(j, slot):
            pltpu.async_copy(rows.at[slot], acc.at[dstv.at[j]], ssems[slot],
                             add=True)

        def wait_scatter(j, slot):
            pltpu.make_async_copy(rows.at[slot], acc.at[dstv.at[j]],
                                  ssems[slot]).wait()

        def do_block(e, cb):
            zero_acc()
            plsc.subcore_barrier()
            start_gather(e, 0, 0)
            start_gather(e, 1, 1)

            def chunk_body(j, _):
                for k in range(4):
                    c = 4 * j + k
                    wait_gather(e, c, k)
                    start_scatter(c, k)
                    sd = (k + 2) % 4
                    if k < 2:
                        @pl.when(j > 0)
                        def _():
                            wait_scatter(c - 2, sd)
                            start_gather(e, c + 2, sd)

                        @pl.when(j == 0)
                        def _():
                            start_gather(e, c + 2, sd)
                    else:
                        @pl.when(j < NCHUNK // 4 - 1)
                        def _():
                            wait_scatter(c - 2, sd)
                            start_gather(e, c + 2, sd)
                return 0
            lax.fori_loop(0, NCHUNK // 4, chunk_body, 0)
            wait_scatter(NCHUNK - 4, 0)
            wait_scatter(NCHUNK - 3, 1)
            wait_scatter(NCHUNK - 2, 2)
            wait_scatter(NCHUNK - 1, 3)
            plsc.subcore_barrier()
            pltpu.sync_copy(acc.at[pl.ds(row0, 640)],
                            sums.at[e * ncb + cb, pl.ds(row0, 640)])
            plsc.subcore_barrier()

        for k in range(ncb // 2):
            cb = c * (ncb // 2) + k
            pltpu.sync_copy(src4.at[cb, s], srcT)
            lax.fori_loop(0, nexp, lambda e, _: (do_block(e, cb), 0)[1], 0)

        if with_counts:
            @pl.when(c == 0)
            def _():
                one16 = jnp.ones((16,), f32)

                def oo(i, _):
                    for k in range(cw // 16):
                        rows[0, i, pl.ds(k * 16, 16)] = one16
                    return 0
                lax.fori_loop(0, 128, oo, 0)
                zero_acc()
                plsc.subcore_barrier()

                def cnt_body(i, _):
                    pltpu.sync_copy(rows.at[0], acc.at[dstv.at[i]], add=True)
                    return 0
                lax.fori_loop(0, NCHUNK, cnt_body, 0)
                plsc.subcore_barrier()
                pltpu.sync_copy(acc.at[pl.ds(row0, 640)],
                                deg.at[pl.ds(row0, 640)])

    kern = pl.kernel(
        body,
        out_type=out_type,
        mesh=mesh,
        compiler_params=pltpu.CompilerParams(use_tc_tiling_on_sc=False),
        scratch_types=[
            pltpu.VMEM((NCHUNK, 128), i32),
            pltpu.VMEM((NCHUNK, 128), i32),
            pltpu.VMEM((4, CHUNK, cw), dt),
            pltpu.VMEM((64, cw), dt),
            pltpu.VMEM_SHARED((NP, cw), dt),
            pltpu.SemaphoreType.DMA,
            pltpu.SemaphoreType.DMA,
            pltpu.SemaphoreType.DMA,
            pltpu.SemaphoreType.DMA,
            pltpu.SemaphoreType.DMA,
            pltpu.SemaphoreType.DMA,
            pltpu.SemaphoreType.DMA,
            pltpu.SemaphoreType.DMA,
        ],
    )
    return kern


def _counts_body(batch_ref, sf_ref):
    b = batch_ref[...]
    oh = (b == lax.broadcasted_iota(i32, (NP, G), 1)).astype(f32)
    counts = jnp.sum(oh, axis=0, keepdims=True)
    lg = jnp.log1p(counts)
    sf_ref[...] = jnp.sum(oh * lg, axis=1, keepdims=True)


def _enc_router_body(x_ref, sf_ref, Wenc_ref, benc_ref, Wr1_ref, wr1s_ref,
                     br1_ref, Wr2_ref, br2_ref, h_ref, probs_ref):
    h = jnp.maximum(jnp.dot(x_ref[...], Wenc_ref[...],
                            preferred_element_type=f32) + benc_ref[0], 0.0)
    r = jnp.dot(h, Wr1_ref[...], preferred_element_type=f32)
    r = jnp.maximum(r + sf_ref[...] * wr1s_ref[0] + br1_ref[0], 0.0)
    lg = jnp.dot(r, Wr2_ref[...], preferred_element_type=f32) + br2_ref[0]
    m = jnp.max(lg, axis=1, keepdims=True)
    p = jnp.exp(lg - m)
    probs_ref[...] = p / jnp.sum(p, axis=1, keepdims=True)
    h_ref[...] = h


def _layer_body(h_ref, A_ref, deg_ref, Ws_ref, Wn_ref, b_ref, *refs, relu,
                fuse_next):
    dinv = 1.0 / jnp.maximum(deg_ref[:, 0:1], 1.0)
    z = _bdot(h_ref[0], Ws_ref[0])
    a = jnp.concatenate([A_ref[0, cb] for cb in range(NCB)], axis=1) * dinv
    z = z + _bdot(a, Wn_ref[0])
    z = z + b_ref[0]
    hh = jnp.maximum(z, 0.0) if relu else z
    if fuse_next:
        Wno_ref, o_ref, p_ref = refs
        o_ref[0] = hh
        p_ref[0] = _bdot(hh, Wno_ref[0]).astype(bf16)
    else:
        (o_ref,) = refs
        o_ref[0] = hh


def _out_body(h_ref, A_ref, deg_ref, probs_ref, Ws_ref, b_ref, o_ref):
    e = pl.program_id(1)
    dinv = 1.0 / jnp.maximum(deg_ref[:, 0:1], 1.0)
    z = _bdot(h_ref[0], Ws_ref[0])
    z = z + jnp.concatenate(
        [A_ref[0, cb].astype(f32) for cb in range(A_ref.shape[1])],
        axis=1) * dinv
    z = z + b_ref[0]
    oh = (lax.broadcasted_iota(i32, (1, NE), 1) == e).astype(f32)
    pe = jnp.sum(probs_ref[...] * oh, axis=1, keepdims=True)

    @pl.when(e == 0)
    def _():
        o_ref[...] = jnp.zeros_like(o_ref)
    o_ref[...] += pe * z


def _counts_call(batch2):
    return pl.pallas_call(
        _counts_body,
        out_shape=jax.ShapeDtypeStruct((NP, 1), f32),
    )(batch2)


def _enc_router_call(x2, sf, Wenc, benc, Wr1h, wr1s, br1, Wr2, br2):
    return pl.pallas_call(
        _enc_router_body,
        grid=(NT,),
        in_specs=[
            pl.BlockSpec((BM, 8), lambda t: (t, 0)),
            pl.BlockSpec((BM, 1), lambda t: (t, 0)),
            pl.BlockSpec((8, H), lambda t: (0, 0)),
            pl.BlockSpec((1, 1, H), lambda t: (0, 0, 0)),
            pl.BlockSpec((H, H), lambda t: (0, 0)),
            pl.BlockSpec((1, 1, H), lambda t: (0, 0, 0)),
            pl.BlockSpec((1, 1, H), lambda t: (0, 0, 0)),
            pl.BlockSpec((H, NE), lambda t: (0, 0)),
            pl.BlockSpec((1, 1, NE), lambda t: (0, 0, 0)),
        ],
        out_specs=[
            pl.BlockSpec((BM, H), lambda t: (t, 0)),
            pl.BlockSpec((BM, NE), lambda t: (t, 0)),
        ],
        out_shape=[
            jax.ShapeDtypeStruct((NP, H), f32),
            jax.ShapeDtypeStruct((NP, NE), f32),
        ],
    )(x2, sf, Wenc, benc, Wr1h, wr1s, br1, Wr2, br2)


def _layer_call(h, A, deg, Ws, Wn, b, *, shared_h, Wno=None):
    h3 = h[None] if shared_h else h
    hmap = (lambda e, t: (0, t, 0)) if shared_h else (lambda e, t: (e, t, 0))
    A4 = A.reshape(-1, NCB, NP, CW)
    amap = (lambda e, t: (0, 0, t, 0)) if A4.shape[0] == 1 else (lambda e, t: (e, 0, t, 0))
    fuse = Wno is not None
    in_specs = [
        pl.BlockSpec((1, BM, H), hmap),
        pl.BlockSpec((1, NCB, BM, CW), amap),
        pl.BlockSpec((BM, CW), lambda e, t: (t, 0)),
        pl.BlockSpec((1, H, H), lambda e, t: (e, 0, 0)),
        pl.BlockSpec((1, H, H), lambda e, t: (e, 0, 0)),
        pl.BlockSpec((1, 1, H), lambda e, t: (e, 0, 0)),
    ]
    args = [h3, A4, deg, Ws, Wn, b]
    out_specs = [pl.BlockSpec((1, BM, H), lambda e, t: (e, t, 0))]
    out_shape = [jax.ShapeDtypeStruct((NE, NP, H), f32)]
    if fuse:
        in_specs.append(pl.BlockSpec((1, H, OUT), lambda e, t: (e, 0, 0)))
        args.append(Wno)
        out_specs.append(pl.BlockSpec((1, BM, OUT), lambda e, t: (e, t, 0)))
        out_shape.append(jax.ShapeDtypeStruct((NE, NP, OUT), bf16))
    res = pl.pallas_call(
        functools.partial(_layer_body, relu=True, fuse_next=fuse),
        grid=(NE, NT),
        in_specs=in_specs,
        out_specs=out_specs,
        out_shape=out_shape,
    )(*args)
    return res if fuse else res[0]


def _out_call(h2, A2p, deg, probs, Wso, bo):
    A4 = A2p.reshape(NE, OUT // CW, NP, CW)
    return pl.pallas_call(
        _out_body,
        grid=(NT, NE),
        in_specs=[
            pl.BlockSpec((1, BM, H), lambda t, e: (e, t, 0)),
            pl.BlockSpec((1, OUT // CW, BM, CW), lambda t, e: (e, 0, t, 0)),
            pl.BlockSpec((BM, CW), lambda t, e: (t, 0)),
            pl.BlockSpec((BM, NE), lambda t, e: (t, 0)),
            pl.BlockSpec((1, H, OUT), lambda t, e: (e, 0, 0)),
            pl.BlockSpec((1, 1, OUT), lambda t, e: (e, 0, 0)),
        ],
        out_specs=pl.BlockSpec((BM, OUT), lambda t, e: (t, 0)),
        out_shape=jax.ShapeDtypeStruct((NP, OUT), f32),
    )(h2, A4, deg, probs, Wso, bo)


@jax.jit
def _forward(x, edge_index, batch, W_enc, b_enc, Wr1, br1, Wr2, br2,
             Ws_h, Wn_h, b_h, Ws_o, Wn_o, b_o):
    src = edge_index[0]
    dst = edge_index[1]

    x2 = jnp.zeros((NP, 8), f32).at[:N, :6].set(x)
    batch2 = jnp.full((NP, 1), G, i32).at[:N, 0].set(batch)
    src_p = jnp.zeros((EPAD,), i32).at[:E].set(src)
    dst_p = jnp.full((EPAD,), N, i32).at[:E].set(dst)
    srcT = src_p.reshape(16, NCHUNK, 128)
    src4 = srcT[None] * NCB + jnp.arange(NCB, dtype=i32)[:, None, None, None]
    src4o = srcT[None] * (OUT // CW) + jnp.arange(OUT // CW, dtype=i32)[:, None, None, None]
    dstT = dst_p.reshape(16, NCHUNK, 128)

    Wenc = jnp.zeros((8, H), f32).at[:6].set(W_enc)
    benc = b_enc.reshape(1, 1, H)
    Wr1h = Wr1[:H]
    wr1s = Wr1[H].reshape(1, 1, H)
    br1r = br1.reshape(1, 1, H)
    br2r = br2.reshape(1, 1, NE)
    Ws0, Ws1 = Ws_h[:, 0], Ws_h[:, 1]
    Wn0, Wn1 = Wn_h[:, 0], Wn_h[:, 1]
    b0 = b_h[:, 0].reshape(NE, 1, H)
    b1 = b_h[:, 1].reshape(NE, 1, H)
    bo = b_o.reshape(NE, 1, OUT)

    agg1 = _make_agg(1, NCB, True)
    agg8 = _make_agg(NE, NCB, False)
    agg8o = _make_agg(NE, OUT // CW, False, dt=bf16)

    sf = _counts_call(batch2)
    h, probs = _enc_router_call(x2, sf, Wenc, benc, Wr1h, wr1s, br1r, Wr2, br2r)

    hv = h.reshape(1, NP * NCB, CW)
    A0, deg = agg1(hv, src4, dstT)
    h1 = _layer_call(h, A0, deg, Ws0, Wn0, b0, shared_h=True)

    h1v = h1.reshape(NE, NP * NCB, CW)
    (A1,) = agg8(h1v, src4, dstT)
    h2, p2 = _layer_call(h1, A1, deg, Ws1, Wn1, b1, shared_h=False, Wno=Wn_o)

    p2v = p2.reshape(NE, NP * (OUT // CW), CW)
    (A2p,) = agg8o(p2v, src4o, dstT)
    out = _out_call(h2, A2p, deg, probs, Ws_o, bo)
    return out[:N]


def kernel(x, edge_index, batch, W_enc, b_enc, Wr1, br1, Wr2, br2,
           Ws_h, Wn_h, b_h, Ws_o, Wn_o, b_o):
    return _forward(x, edge_index, batch, W_enc, b_enc, Wr1, br1, Wr2, br2,
                    Ws_h, Wn_h, b_h, Ws_o, Wn_o, b_o)


# speedup vs baseline: 1.4530x; 1.3114x over previous
import functools

import jax
import jax.numpy as jnp
from jax import lax
from jax.experimental import pallas as pl
from jax.experimental.pallas import tpu as pltpu
from jax.experimental.pallas import tpu_sc as plsc

N = 10000
NP = 10240
E = 160000
EPT = 10240
EPAD = 16 * EPT
CHUNK = 128
NCHUNK = EPT // CHUNK
H = 512
OUT = 256
NE = 8
G = 64
BM = 640
NT = NP // BM
CW = 64
NCB = 8

f32 = jnp.float32
bf16 = jnp.bfloat16
i32 = jnp.int32


def _bdot(a, b):
    return jnp.dot(a.astype(bf16), b.astype(bf16), preferred_element_type=f32)


def _make_agg(nexp: int, ncb: int, with_counts: bool, cw: int = CW,
              dt=f32):
    mesh = plsc.VectorSubcoreMesh(core_axis_name="c", subcore_axis_name="s")
    out_type = [jax.ShapeDtypeStruct((nexp * ncb, NP, cw), dt)]
    if with_counts:
        out_type.append(jax.ShapeDtypeStruct((NP, cw), f32))

    def body(hv, src4, dstT, *refs):
        if with_counts:
            (sums, deg, srcT, dstv, rows, zbuf, acc,
             gs0, gs1, gs2, gs3, ss0, ss1, ss2, ss3) = refs
        else:
            (sums, srcT, dstv, rows, zbuf, acc,
             gs0, gs1, gs2, gs3, ss0, ss1, ss2, ss3) = refs
        gsems = (gs0, gs1, gs2, gs3)
        ssems = (ss0, ss1, ss2, ss3)
        c = lax.axis_index("c")
        s = lax.axis_index("s")
        row0 = s * 640

        lanes = 16 if dt == f32 else 32
        zv = jnp.zeros((lanes,), dt)

        def zz(i, _):
            for k in range(cw // lanes):
                zbuf[i, pl.ds(k * lanes, lanes)] = zv
            return 0
        lax.fori_loop(0, 64, zz, 0)

        pltpu.sync_copy(dstT.at[s], dstv)

        def zero_acc():
            for m in range(10):
                pltpu.async_copy(zbuf, acc.at[pl.ds(row0 + 64 * m, 64)], gs0)
            for m in range(10):
                pltpu.make_async_copy(zbuf, acc.at[pl.ds(row0 + 64 * m, 64)],
                                      gs0).wait()

        def start_gather(e, j, slot):
            pltpu.async_copy(hv.at[e].at[srcT.at[j]], rows.at[slot], gsems[slot])

        def wait_gather(e, j, slot):
            pltpu.make_async_copy(hv.at[e].at[srcT.at[j]], rows.at[slot],
                                  gsems[slot]).wait()

        def start_scatter(j, slot):
            pltpu.async_copy(rows.at[slot], acc.at[dstv.at[j]], ssems[slot],
                             add=True)

        def wait_scatter(j, slot):
            pltpu.make_async_copy(rows.at[slot], acc.at[dstv.at[j]],
                                  ssems[slot]).wait()

        def do_block(e, cb):
            zero_acc()
            plsc.subcore_barrier()
            start_gather(e, 0, 0)
            start_gather(e, 1, 1)

            def chunk_body(j, _):
                for k in range(4):
                    c = 4 * j + k
                    wait_gather(e, c, k)
                    start_scatter(c, k)
                    sd = (k + 2) % 4
                    if k < 2:
                        @pl.when(j > 0)
                        def _():
                            wait_scatter(c - 2, sd)
                            start_gather(e, c + 2, sd)

                        @pl.when(j == 0)
                        def _():
                            start_gather(e, c + 2, sd)
                    else:
                        @pl.when(j < NCHUNK // 4 - 1)
                        def _():
                            wait_scatter(c - 2, sd)
                            start_gather(e, c + 2, sd)
                return 0
            lax.fori_loop(0, NCHUNK // 4, chunk_body, 0)
            wait_scatter(NCHUNK - 4, 0)
            wait_scatter(NCHUNK - 3, 1)
            wait_scatter(NCHUNK - 2, 2)
            wait_scatter(NCHUNK - 1, 3)
            plsc.subcore_barrier()
            pltpu.sync_copy(acc.at[pl.ds(row0, 640)],
                            sums.at[e * ncb + cb, pl.ds(row0, 640)])
            plsc.subcore_barrier()

        for k in range(ncb // 2):
            cb = c * (ncb // 2) + k
            pltpu.sync_copy(src4.at[cb, s], srcT)
            lax.fori_loop(0, nexp, lambda e, _: (do_block(e, cb), 0)[1], 0)

        if with_counts:
            @pl.when(c == 0)
            def _():
                one16 = jnp.ones((16,), f32)

                def oo(i, _):
                    for k in range(cw // 16):
                        rows[0, i, pl.ds(k * 16, 16)] = one16
                    return 0
                lax.fori_loop(0, 128, oo, 0)
                zero_acc()
                plsc.subcore_barrier()

                def cnt_body(i, _):
                    pltpu.sync_copy(rows.at[0], acc.at[dstv.at[i]], add=True)
                    return 0
                lax.fori_loop(0, NCHUNK, cnt_body, 0)
                plsc.subcore_barrier()
                pltpu.sync_copy(acc.at[pl.ds(row0, 640)],
                                deg.at[pl.ds(row0, 640)])

    kern = pl.kernel(
        body,
        out_type=out_type,
        mesh=mesh,
        compiler_params=pltpu.CompilerParams(use_tc_tiling_on_sc=False),
        scratch_types=[
            pltpu.VMEM((NCHUNK, 128), i32),
            pltpu.VMEM((NCHUNK, 128), i32),
            pltpu.VMEM((4, CHUNK, cw), dt),
            pltpu.VMEM((64, cw), dt),
            pltpu.VMEM_SHARED((NP, cw), dt),
            pltpu.SemaphoreType.DMA,
            pltpu.SemaphoreType.DMA,
            pltpu.SemaphoreType.DMA,
            pltpu.SemaphoreType.DMA,
            pltpu.SemaphoreType.DMA,
            pltpu.SemaphoreType.DMA,
            pltpu.SemaphoreType.DMA,
            pltpu.SemaphoreType.DMA,
        ],
    )
    return kern


def _counts_body(batch_ref, sf_ref):
    b = batch_ref[...]
    oh = (b == lax.broadcasted_iota(i32, (NP, G), 1)).astype(f32)
    counts = jnp.sum(oh, axis=0, keepdims=True)
    lg = jnp.log1p(counts)
    sf_ref[...] = jnp.sum(oh * lg, axis=1, keepdims=True)


def _enc_router_body(x_ref, sf_ref, Wenc_ref, benc_ref, Wr1_ref, wr1s_ref,
                     br1_ref, Wr2_ref, br2_ref, h_ref, probs_ref):
    h = jnp.maximum(jnp.dot(x_ref[...], Wenc_ref[...],
                            preferred_element_type=f32) + benc_ref[0], 0.0)
    r = jnp.dot(h, Wr1_ref[...], preferred_element_type=f32)
    r = jnp.maximum(r + sf_ref[...] * wr1s_ref[0] + br1_ref[0], 0.0)
    lg = jnp.dot(r, Wr2_ref[...], preferred_element_type=f32) + br2_ref[0]
    m = jnp.max(lg, axis=1, keepdims=True)
    p = jnp.exp(lg - m)
    probs_ref[...] = p / jnp.sum(p, axis=1, keepdims=True)
    h_ref[...] = h


def _layer_body(h_ref, A_ref, deg_ref, Ws_ref, Wn_ref, b_ref, *refs, relu,
                fuse_next, emit_bf):
    dinv = 1.0 / jnp.maximum(deg_ref[:, 0:1], 1.0)
    z = _bdot(h_ref[0], Ws_ref[0])
    a = jnp.concatenate([A_ref[0, cb].astype(f32) for cb in range(NCB)],
                        axis=1) * dinv
    z = z + _bdot(a, Wn_ref[0])
    z = z + b_ref[0]
    hh = jnp.maximum(z, 0.0) if relu else z
    if fuse_next:
        Wno_ref, o_ref, p_ref = refs
        o_ref[0] = hh
        p_ref[0] = _bdot(hh, Wno_ref[0]).astype(bf16)
    elif emit_bf:
        o_ref, obf_ref = refs
        o_ref[0] = hh
        obf_ref[0] = hh.astype(bf16)
    else:
        (o_ref,) = refs
        o_ref[0] = hh


def _out_body(h_ref, A_ref, deg_ref, probs_ref, Ws_ref, b_ref, o_ref):
    e = pl.program_id(1)
    dinv = 1.0 / jnp.maximum(deg_ref[:, 0:1], 1.0)
    z = _bdot(h_ref[0], Ws_ref[0])
    z = z + jnp.concatenate(
        [A_ref[0, cb].astype(f32) for cb in range(A_ref.shape[1])],
        axis=1) * dinv
    z = z + b_ref[0]
    oh = (lax.broadcasted_iota(i32, (1, NE), 1) == e).astype(f32)
    pe = jnp.sum(probs_ref[...] * oh, axis=1, keepdims=True)

    @pl.when(e == 0)
    def _():
        o_ref[...] = jnp.zeros_like(o_ref)
    o_ref[...] += pe * z


def _counts_call(batch2):
    return pl.pallas_call(
        _counts_body,
        out_shape=jax.ShapeDtypeStruct((NP, 1), f32),
    )(batch2)


def _enc_router_call(x2, sf, Wenc, benc, Wr1h, wr1s, br1, Wr2, br2):
    return pl.pallas_call(
        _enc_router_body,
        grid=(NT,),
        in_specs=[
            pl.BlockSpec((BM, 8), lambda t: (t, 0)),
            pl.BlockSpec((BM, 1), lambda t: (t, 0)),
            pl.BlockSpec((8, H), lambda t: (0, 0)),
            pl.BlockSpec((1, 1, H), lambda t: (0, 0, 0)),
            pl.BlockSpec((H, H), lambda t: (0, 0)),
            pl.BlockSpec((1, 1, H), lambda t: (0, 0, 0)),
            pl.BlockSpec((1, 1, H), lambda t: (0, 0, 0)),
            pl.BlockSpec((H, NE), lambda t: (0, 0)),
            pl.BlockSpec((1, 1, NE), lambda t: (0, 0, 0)),
        ],
        out_specs=[
            pl.BlockSpec((BM, H), lambda t: (t, 0)),
            pl.BlockSpec((BM, NE), lambda t: (t, 0)),
        ],
        out_shape=[
            jax.ShapeDtypeStruct((NP, H), f32),
            jax.ShapeDtypeStruct((NP, NE), f32),
        ],
    )(x2, sf, Wenc, benc, Wr1h, wr1s, br1, Wr2, br2)


def _layer_call(h, A, deg, Ws, Wn, b, *, shared_h, Wno=None, emit_bf=False):
    h3 = h[None] if shared_h else h
    hmap = (lambda e, t: (0, t, 0)) if shared_h else (lambda e, t: (e, t, 0))
    A4 = A.reshape(-1, NCB, NP, CW)
    amap = (lambda e, t: (0, 0, t, 0)) if A4.shape[0] == 1 else (lambda e, t: (e, 0, t, 0))
    fuse = Wno is not None
    in_specs = [
        pl.BlockSpec((1, BM, H), hmap),
        pl.BlockSpec((1, NCB, BM, CW), amap),
        pl.BlockSpec((BM, CW), lambda e, t: (t, 0)),
        pl.BlockSpec((1, H, H), lambda e, t: (e, 0, 0)),
        pl.BlockSpec((1, H, H), lambda e, t: (e, 0, 0)),
        pl.BlockSpec((1, 1, H), lambda e, t: (e, 0, 0)),
    ]
    args = [h3, A4, deg, Ws, Wn, b]
    out_specs = [pl.BlockSpec((1, BM, H), lambda e, t: (e, t, 0))]
    out_shape = [jax.ShapeDtypeStruct((NE, NP, H), f32)]
    if fuse:
        in_specs.append(pl.BlockSpec((1, H, OUT), lambda e, t: (e, 0, 0)))
        args.append(Wno)
        out_specs.append(pl.BlockSpec((1, BM, OUT), lambda e, t: (e, t, 0)))
        out_shape.append(jax.ShapeDtypeStruct((NE, NP, OUT), bf16))
    elif emit_bf:
        out_specs.append(pl.BlockSpec((1, BM, H), lambda e, t: (e, t, 0)))
        out_shape.append(jax.ShapeDtypeStruct((NE, NP, H), bf16))
    res = pl.pallas_call(
        functools.partial(_layer_body, relu=True, fuse_next=fuse,
                          emit_bf=emit_bf),
        grid=(NE, NT),
        in_specs=in_specs,
        out_specs=out_specs,
        out_shape=out_shape,
    )(*args)
    return res if (fuse or emit_bf) else res[0]


def _out_call(h2, A2p, deg, probs, Wso, bo):
    A4 = A2p.reshape(NE, OUT // CW, NP, CW)
    return pl.pallas_call(
        _out_body,
        grid=(NT, NE),
        in_specs=[
            pl.BlockSpec((1, BM, H), lambda t, e: (e, t, 0)),
            pl.BlockSpec((1, OUT // CW, BM, CW), lambda t, e: (e, 0, t, 0)),
            pl.BlockSpec((BM, CW), lambda t, e: (t, 0)),
            pl.BlockSpec((BM, NE), lambda t, e: (t, 0)),
            pl.BlockSpec((1, H, OUT), lambda t, e: (e, 0, 0)),
            pl.BlockSpec((1, 1, OUT), lambda t, e: (e, 0, 0)),
        ],
        out_specs=pl.BlockSpec((BM, OUT), lambda t, e: (t, 0)),
        out_shape=jax.ShapeDtypeStruct((NP, OUT), f32),
    )(h2, A4, deg, probs, Wso, bo)


@jax.jit
def _forward(x, edge_index, batch, W_enc, b_enc, Wr1, br1, Wr2, br2,
             Ws_h, Wn_h, b_h, Ws_o, Wn_o, b_o):
    src = edge_index[0]
    dst = edge_index[1]

    x2 = jnp.zeros((NP, 8), f32).at[:N, :6].set(x)
    batch2 = jnp.full((NP, 1), G, i32).at[:N, 0].set(batch)
    src_p = jnp.zeros((EPAD,), i32).at[:E].set(src)
    dst_p = jnp.full((EPAD,), N, i32).at[:E].set(dst)
    srcT = src_p.reshape(16, NCHUNK, 128)
    src4 = srcT[None] * NCB + jnp.arange(NCB, dtype=i32)[:, None, None, None]
    src4o = srcT[None] * (OUT // CW) + jnp.arange(OUT // CW, dtype=i32)[:, None, None, None]
    dstT = dst_p.reshape(16, NCHUNK, 128)

    Wenc = jnp.zeros((8, H), f32).at[:6].set(W_enc)
    benc = b_enc.reshape(1, 1, H)
    Wr1h = Wr1[:H]
    wr1s = Wr1[H].reshape(1, 1, H)
    br1r = br1.reshape(1, 1, H)
    br2r = br2.reshape(1, 1, NE)
    Ws0, Ws1 = Ws_h[:, 0], Ws_h[:, 1]
    Wn0, Wn1 = Wn_h[:, 0], Wn_h[:, 1]
    b0 = b_h[:, 0].reshape(NE, 1, H)
    b1 = b_h[:, 1].reshape(NE, 1, H)
    bo = b_o.reshape(NE, 1, OUT)

    agg1 = _make_agg(1, NCB, True)
    agg8 = _make_agg(NE, NCB, False, dt=bf16)
    agg8o = _make_agg(NE, OUT // CW, False, dt=bf16)

    sf = _counts_call(batch2)
    h, probs = _enc_router_call(x2, sf, Wenc, benc, Wr1h, wr1s, br1r, Wr2, br2r)

    hv = h.reshape(1, NP * NCB, CW)
    A0, deg = agg1(hv, src4, dstT)
    h1, h1bf = _layer_call(h, A0, deg, Ws0, Wn0, b0, shared_h=True,
                           emit_bf=True)

    h1v = h1bf.reshape(NE, NP * NCB, CW)
    (A1,) = agg8(h1v, src4, dstT)
    h2, p2 = _layer_call(h1, A1, deg, Ws1, Wn1, b1, shared_h=False, Wno=Wn_o)

    p2v = p2.reshape(NE, NP * (OUT // CW), CW)
    (A2p,) = agg8o(p2v, src4o, dstT)
    out = _out_call(h2, A2p, deg, probs, Ws_o, bo)
    return out[:N]


def kernel(x, edge_index, batch, W_enc, b_enc, Wr1, br1, Wr2, br2,
           Ws_h, Wn_h, b_h, Ws_o, Wn_o, b_o):
    return _forward(x, edge_index, batch, W_enc, b_enc, Wr1, br1, Wr2, br2,
                    Ws_h, Wn_h, b_h, Ws_o, Wn_o, b_o)
